# baseline (speedup 1.0000x reference)
import functools

import jax
import jax.numpy as jnp
from jax.experimental import pallas as pl
from jax.experimental.pallas import tpu as pltpu

N_EXPERTS = 64
TOP_K = 8
N_GROUP = 8
TOPK_GROUP = 4
SCALE = 2.5
PER_GROUP = N_EXPERTS // N_GROUP

BT = 2048


def _gate_block(x1_ref, x2_ref, w_ref, b_ref, w_out_ref, i_out_ref):
    bt = x1_ref.shape[0] * 2
    neg_inf = jnp.float32(-jnp.inf)

    logits = jnp.concatenate(
        [
            jax.lax.dot_general(
                w_ref[...], x_ref[...],
                dimension_numbers=(((1,), (1,)), ((), ())),
                preferred_element_type=jnp.float32,
            )
            for x_ref in (x1_ref, x2_ref)
        ],
        axis=1,
    )
    scores = jax.nn.sigmoid(logits)
    sfc = scores + b_ref[...]

    iota_e = jax.lax.broadcasted_iota(jnp.int32, (N_EXPERTS, bt), 0)
    iota_g8 = jax.lax.broadcasted_iota(jnp.int32, (PER_GROUP, bt), 0)

    gs_rows = []
    for g in range(N_GROUP):
        sub = sfc[g * PER_GROUP : (g + 1) * PER_GROUP, :]
        m1 = jnp.max(sub, axis=0, keepdims=True)
        am1 = jnp.min(jnp.where(sub == m1, iota_g8, PER_GROUP), axis=0, keepdims=True)
        m2 = jnp.max(jnp.where(iota_g8 == am1, neg_inf, sub), axis=0, keepdims=True)
        gs_rows.append(m1 + m2)
    gs8 = jnp.concatenate(gs_rows, axis=0)

    iota_grp = jax.lax.broadcasted_iota(jnp.int32, (N_GROUP, bt), 0)
    gmask8 = jnp.zeros((N_GROUP, bt), jnp.bool_)
    work_g = gs8
    for _ in range(TOPK_GROUP):
        m = jnp.max(work_g, axis=0, keepdims=True)
        amg = jnp.min(jnp.where(work_g == m, iota_grp, N_GROUP), axis=0, keepdims=True)
        sel = iota_grp == amg
        gmask8 = gmask8 | sel
        work_g = jnp.where(sel, neg_inf, work_g)

    tmp = jnp.concatenate(
        [
            jnp.where(
                gmask8[g : g + 1, :],
                sfc[g * PER_GROUP : (g + 1) * PER_GROUP, :],
                0.0,
            )
            for g in range(N_GROUP)
        ],
        axis=0,
    )

    vals = []
    idxs = []
    work = tmp
    for k in range(TOP_K):
        m = jnp.max(work, axis=0, keepdims=True)
        am = jnp.min(jnp.where(work == m, iota_e, N_EXPERTS), axis=0, keepdims=True)
        hit = iota_e == am
        wv = jnp.max(jnp.where(hit, scores, neg_inf), axis=0, keepdims=True)
        vals.append(wv)
        idxs.append(am)
        work = jnp.where(hit, neg_inf, work)

    denom = vals[0]
    for v in vals[1:]:
        denom = denom + v
    denom = denom + 1e-20
    w_out_ref[...] = jnp.concatenate(vals, axis=0) / denom * SCALE
    i_out_ref[...] = jnp.concatenate(idxs, axis=0)


@jax.jit
def _gate(hidden_states, weight, bias2d):
    s, d = hidden_states.shape
    grid = (s // BT,)
    w_t, i_t = pl.pallas_call(
        _gate_block,
        grid=grid,
        in_specs=[
            pl.BlockSpec((BT // 2, d), lambda i: (2 * i, 0)),
            pl.BlockSpec((BT // 2, d), lambda i: (2 * i + 1, 0)),
            pl.BlockSpec((N_EXPERTS, d), lambda i: (0, 0)),
            pl.BlockSpec((N_EXPERTS, 1), lambda i: (0, 0)),
        ],
        out_specs=[
            pl.BlockSpec((TOP_K, BT), lambda i: (0, i)),
            pl.BlockSpec((TOP_K, BT), lambda i: (0, i)),
        ],
        out_shape=[
            jax.ShapeDtypeStruct((TOP_K, s), jnp.float32),
            jax.ShapeDtypeStruct((TOP_K, s), jnp.int32),
        ],
    )(hidden_states, hidden_states, weight, bias2d)
    return w_t.T, i_t.T


def kernel(hidden_states, weight, e_score_correction_bias):
    bias2d = e_score_correction_bias.reshape(N_EXPERTS, 1)
    topk_weight, topk_idx = _gate(hidden_states, weight, bias2d)
    return (topk_weight, topk_idx)


# speedup vs baseline: 1.0100x; 1.0100x over previous
import jax
import jax.numpy as jnp
from jax.experimental import pallas as pl

N_EXPERTS = 64
TOP_K = 8
N_GROUP = 8
TOPK_GROUP = 4
SCALE = 2.5
PER_GROUP = N_EXPERTS // N_GROUP

BT = 2048


def _gate_block(x_ref, w_ref, b_ref, w_out_ref, i_out_ref):
    bt = x_ref.shape[0]
    neg_inf = jnp.float32(-jnp.inf)

    logits = jax.lax.dot_general(
        w_ref[...], x_ref[...],
        dimension_numbers=(((1,), (1,)), ((), ())),
        preferred_element_type=jnp.float32,
    )
    scores = jax.nn.sigmoid(logits)
    sfc = scores + b_ref[...]

    iota_e = jax.lax.broadcasted_iota(jnp.int32, (N_EXPERTS, bt), 0)
    iota_g8 = jax.lax.broadcasted_iota(jnp.int32, (PER_GROUP, bt), 0)

    gs_rows = []
    for g in range(N_GROUP):
        sub = sfc[g * PER_GROUP : (g + 1) * PER_GROUP, :]
        m1 = jnp.max(sub, axis=0, keepdims=True)
        am1 = jnp.min(jnp.where(sub == m1, iota_g8, PER_GROUP), axis=0, keepdims=True)
        m2 = jnp.max(jnp.where(iota_g8 == am1, neg_inf, sub), axis=0, keepdims=True)
        gs_rows.append(m1 + m2)
    gs8 = jnp.concatenate(gs_rows, axis=0)

    iota_grp = jax.lax.broadcasted_iota(jnp.int32, (N_GROUP, bt), 0)
    gmask8 = jnp.zeros((N_GROUP, bt), jnp.bool_)
    work_g = gs8
    for _ in range(TOPK_GROUP):
        m = jnp.max(work_g, axis=0, keepdims=True)
        amg = jnp.min(jnp.where(work_g == m, iota_grp, N_GROUP), axis=0, keepdims=True)
        sel = iota_grp == amg
        gmask8 = gmask8 | sel
        work_g = jnp.where(sel, neg_inf, work_g)

    tmp = jnp.concatenate(
        [
            jnp.where(
                gmask8[g : g + 1, :],
                sfc[g * PER_GROUP : (g + 1) * PER_GROUP, :],
                0.0,
            )
            for g in range(N_GROUP)
        ],
        axis=0,
    )

    vals = []
    idxs = []
    work = tmp
    for k in range(TOP_K):
        m = jnp.max(work, axis=0, keepdims=True)
        am = jnp.min(jnp.where(work == m, iota_e, N_EXPERTS), axis=0, keepdims=True)
        hit = iota_e == am
        wv = jnp.max(jnp.where(hit, scores, neg_inf), axis=0, keepdims=True)
        vals.append(wv)
        idxs.append(am)
        work = jnp.where(hit, neg_inf, work)

    denom = vals[0]
    for v in vals[1:]:
        denom = denom + v
    denom = denom + 1e-20
    w_out_ref[...] = jnp.concatenate(vals, axis=0) / denom * SCALE
    i_out_ref[...] = jnp.concatenate(idxs, axis=0)


@jax.jit
def _gate(hidden_states, weight, bias2d):
    s, d = hidden_states.shape
    grid = (s // BT,)
    w_t, i_t = pl.pallas_call(
        _gate_block,
        grid=grid,
        in_specs=[
            pl.BlockSpec((BT, d), lambda i: (i, 0)),
            pl.BlockSpec((N_EXPERTS, d), lambda i: (0, 0)),
            pl.BlockSpec((N_EXPERTS, 1), lambda i: (0, 0)),
        ],
        out_specs=[
            pl.BlockSpec((TOP_K, BT), lambda i: (0, i)),
            pl.BlockSpec((TOP_K, BT), lambda i: (0, i)),
        ],
        out_shape=[
            jax.ShapeDtypeStruct((TOP_K, s), jnp.float32),
            jax.ShapeDtypeStruct((TOP_K, s), jnp.int32),
        ],
    )(hidden_states, weight, bias2d)
    return w_t.T, i_t.T


def kernel(hidden_states, weight, e_score_correction_bias):
    bias2d = e_score_correction_bias.reshape(N_EXPERTS, 1)
    topk_weight, topk_idx = _gate(hidden_states, weight, bias2d)
    return (topk_weight, topk_idx)
